# Initial kernel scaffold; baseline (speedup 1.0000x reference)
import jax
import jax.numpy as jnp
from jax.experimental import pallas as pl


def kernel(lengths, table):
    raise NotImplementedError("write your pallas kernel here")


# speedup vs baseline: 3.1050x; 3.1050x over previous
import functools

import jax
import jax.numpy as jnp
from jax import lax
from jax.experimental import pallas as pl
from jax.experimental.pallas import tpu as pltpu
from jax.experimental.pallas import tpu_sc as plsc

_NC = 2
_NS = 16
_NW = _NC * _NS

_Q = 4
_NQUAD = 9 ** _Q
_ROW = 128
_BQ = (_NQUAD + _NS - 1) // _NS


@functools.lru_cache(maxsize=None)
def _build(n_rows: int, dim: int):
    qdim = _Q * dim
    assert n_rows % _NW == 0
    r_per_w = n_rows // _NW
    mesh = plsc.VectorSubcoreMesh(core_axis_name="c", subcore_axis_name="s")

    @functools.partial(
        pl.kernel,
        mesh=mesh,
        out_type=jax.ShapeDtypeStruct((n_rows, _ROW, qdim), jnp.float32),
        scratch_types=[
            pltpu.VMEM((9, dim), jnp.float32),
            pltpu.VMEM((_BQ, qdim), jnp.float32),
            pltpu.VMEM((_Q * _ROW,), jnp.int32),
            pltpu.VMEM((_ROW,), jnp.int32),
            pltpu.VMEM((_ROW, qdim), jnp.float32),
            pltpu.VMEM_SHARED((_NS * _BQ, qdim), jnp.float32),
            pltpu.SemaphoreType.DMA,
        ],
        compiler_params=pltpu.CompilerParams(
            use_tc_tiling_on_sc=False, needs_layout_passes=False),
    )
    def gather_kernel(idx_hbm, table_hbm, out_hbm,
                      tab_v, stage_v, ibuf, qbuf, obuf, tab_sh, sem):
        cid = lax.axis_index("c")
        sid = lax.axis_index("s")
        wid = sid * _NC + cid

        pltpu.sync_copy(table_hbm, tab_v)
        lane = lax.iota(jnp.int32, 16)
        q0 = sid * _BQ

        def build(ql, carry):
            q = jnp.minimum(q0 + ql, _NQUAD - 1)
            qv = jnp.full((16,), 0, jnp.int32) + q
            d0 = qv // 729
            r0 = qv - d0 * 729
            d1 = r0 // 81
            r1 = r0 - d1 * 81
            d2 = r1 // 9
            d3 = r1 - d2 * 9
            for s in range(qdim // 16):
                p = s * 16 + lane
                e = p // dim
                o = p - e * dim
                row = jnp.where(e == 0, d0,
                      jnp.where(e == 1, d1,
                      jnp.where(e == 2, d2, d3)))
                stage_v[ql, pl.ds(s * 16, 16)] = plsc.load_gather(
                    tab_v, [row, o])
            return carry

        lax.fori_loop(0, _BQ, build, 0)
        pltpu.sync_copy(stage_v, tab_sh.at[pl.ds(q0, _BQ)])
        plsc.subcore_barrier()

        base4 = lane * _Q

        def body(r, carry):
            row = wid * r_per_w + r
            pltpu.sync_copy(idx_hbm.at[row], ibuf)
            for v in range(_ROW // 16):
                g0 = plsc.load_gather(ibuf, [base4 + (64 * v + 0)])
                g1 = plsc.load_gather(ibuf, [base4 + (64 * v + 1)])
                g2 = plsc.load_gather(ibuf, [base4 + (64 * v + 2)])
                g3 = plsc.load_gather(ibuf, [base4 + (64 * v + 3)])
                qbuf[pl.ds(16 * v, 16)] = ((g0 * 9 + g1) * 9 + g2) * 9 + g3
            pltpu.async_copy(tab_sh.at[qbuf], obuf, sem).wait()
            pltpu.sync_copy(obuf, out_hbm.at[row])
            return carry

        lax.fori_loop(0, r_per_w, body, 0)

    return gather_kernel


def kernel(lengths, table):
    n, s = lengths.shape
    _, dim = table.shape
    m = n * s
    n_rows = m // (_Q * _ROW)
    idx = lengths.reshape(n_rows, _Q * _ROW)
    out = _build(n_rows, dim)(idx, table)
    return out.reshape(n, s, dim)


# speedup vs baseline: 3.2486x; 1.0463x over previous
import functools

import jax
import jax.numpy as jnp
from jax import lax
from jax.experimental import pallas as pl
from jax.experimental.pallas import tpu as pltpu
from jax.experimental.pallas import tpu_sc as plsc

_NC = 2
_NS = 16
_NW = _NC * _NS

_Q = 4
_NQUAD = 9 ** _Q
_ROW = 128
_BQ = (_NQUAD + _NS - 1) // _NS
_NB = 4


@functools.lru_cache(maxsize=None)
def _build(n_rows: int, dim: int):
    qdim = _Q * dim
    assert n_rows % (_NW * _NB) == 0
    r_per_w = n_rows // _NW
    n_groups = r_per_w // _NB
    mesh = plsc.VectorSubcoreMesh(core_axis_name="c", subcore_axis_name="s")

    @functools.partial(
        pl.kernel,
        mesh=mesh,
        out_type=jax.ShapeDtypeStruct((n_rows, _ROW, qdim), jnp.float32),
        scratch_types=[
            pltpu.VMEM((9, dim), jnp.float32),
            pltpu.VMEM((_BQ, qdim), jnp.float32),
            [pltpu.VMEM((_Q * _ROW,), jnp.int32)] * _NB,
            [pltpu.VMEM((_ROW,), jnp.int32)] * _NB,
            [pltpu.VMEM((_ROW, qdim), jnp.float32)] * _NB,
            pltpu.VMEM_SHARED((_NS * _BQ, qdim), jnp.float32),
            pltpu.SemaphoreType.DMA,
            pltpu.SemaphoreType.DMA,
            pltpu.SemaphoreType.DMA,
        ],
        compiler_params=pltpu.CompilerParams(
            use_tc_tiling_on_sc=False, needs_layout_passes=False),
    )
    def gather_kernel(idx_hbm, table_hbm, out_hbm,
                      tab_v, stage_v, ibufs, qbufs, obufs, tab_sh,
                      sem_in, sem_g, sem_out):
        cid = lax.axis_index("c")
        sid = lax.axis_index("s")
        wid = sid * _NC + cid
        row0 = wid * r_per_w

        pltpu.sync_copy(table_hbm, tab_v)
        lane = lax.iota(jnp.int32, 16)
        q0 = sid * _BQ

        def build(ql, carry):
            q = jnp.minimum(q0 + ql, _NQUAD - 1)
            qv = jnp.full((16,), 0, jnp.int32) + q
            d0 = qv // 729
            r0 = qv - d0 * 729
            d1 = r0 // 81
            r1 = r0 - d1 * 81
            d2 = r1 // 9
            d3 = r1 - d2 * 9
            for s in range(qdim // 16):
                p = s * 16 + lane
                e = p // dim
                o = p - e * dim
                row = jnp.where(e == 0, d0,
                      jnp.where(e == 1, d1,
                      jnp.where(e == 2, d2, d3)))
                stage_v[ql, pl.ds(s * 16, 16)] = plsc.load_gather(
                    tab_v, [row, o])
            return carry

        lax.fori_loop(0, _BQ, build, 0)
        pltpu.sync_copy(stage_v, tab_sh.at[pl.ds(q0, _BQ)])
        plsc.subcore_barrier()

        base4 = lane * _Q

        def fold_ids(ibuf, qbuf):
            for v in range(_ROW // 16):
                g0 = plsc.load_gather(ibuf, [base4 + (64 * v + 0)])
                g1 = plsc.load_gather(ibuf, [base4 + (64 * v + 1)])
                g2 = plsc.load_gather(ibuf, [base4 + (64 * v + 2)])
                g3 = plsc.load_gather(ibuf, [base4 + (64 * v + 3)])
                qbuf[pl.ds(16 * v, 16)] = ((g0 * 9 + g1) * 9 + g2) * 9 + g3

        for b in range(_NB):
            pltpu.async_copy(idx_hbm.at[row0 + b], ibufs[b], sem_in)

        def group(g, carry):
            for b in range(_NB):
                r = g * _NB + b
                row = row0 + r
                pltpu.make_async_copy(idx_hbm.at[row], ibufs[b], sem_in).wait()
                fold_ids(ibufs[b], qbufs[b])
                @pl.when(g < n_groups - 1)
                def _():
                    pltpu.async_copy(
                        idx_hbm.at[row + _NB], ibufs[b], sem_in)
                @pl.when(g > 0)
                def _():
                    pltpu.make_async_copy(
                        out_hbm.at[row], obufs[b], sem_out).wait()
                pltpu.async_copy(tab_sh.at[qbufs[b]], obufs[b], sem_g).wait()
                pltpu.async_copy(obufs[b], out_hbm.at[row], sem_out)
            return carry

        lax.fori_loop(0, n_groups, group, 0)
        for b in range(_NB):
            pltpu.make_async_copy(
                out_hbm.at[row0], obufs[b], sem_out).wait()

    return gather_kernel


def kernel(lengths, table):
    n, s = lengths.shape
    _, dim = table.shape
    m = n * s
    n_rows = m // (_Q * _ROW)
    idx = lengths.reshape(n_rows, _Q * _ROW)
    out = _build(n_rows, dim)(idx, table)
    return out.reshape(n, s, dim)
